# baseline (speedup 1.0000x reference)
import functools

import jax
import jax.numpy as jnp
from jax import lax
from jax.experimental import pallas as pl
from jax.experimental.pallas import tpu as pltpu
from jax.experimental.pallas import tpu_sc as plsc

NUM_CLASSES = 100
THRESH = 0.05
B = 16384
R1 = 2048
BIGI32 = 2**30
KEY_BASE = 0x3C000000


def _stage1_body(x_ref, maxp_ref, lab_ref, vk_ref, kv_ref):
    step = pl.program_id(0)
    x = x_ref[...]
    m = jnp.max(x, axis=1, keepdims=True)
    e = jnp.exp(x - m)
    s = jnp.sum(e, axis=1, keepdims=True)
    p = e / s
    maxp = jnp.max(p, axis=1, keepdims=True)
    iotaf = jax.lax.broadcasted_iota(jnp.int32, p.shape, 1).astype(jnp.float32)
    labf = jnp.min(jnp.where(p >= maxp, iotaf, jnp.float32(1e9)),
                   axis=1, keepdims=True)
    lab = labf.astype(jnp.int32)
    conf = maxp >= THRESH
    vk = jnp.where(conf, jax.lax.bitcast_convert_type(maxp, jnp.int32),
                   jnp.int32(-1))
    maxp_ref[...] = maxp
    lab_ref[...] = lab
    vk_ref[...] = vk
    lane = jax.lax.broadcasted_iota(jnp.int32, (R1, 128), 1)
    onehot = (lab == lane) & conf
    cnt = jnp.sum(onehot.astype(jnp.int32), axis=0, keepdims=True)

    @pl.when(step == 0)
    def _():
        kv_ref[...] = cnt

    @pl.when(step > 0)
    def _():
        kv_ref[...] += cnt

    @pl.when(step == pl.num_programs(0) - 1)
    def _():
        counts = kv_ref[...]
        lane1 = jax.lax.broadcasted_iota(jnp.int32, (1, 128), 1)
        valid = lane1 < NUM_CLASSES
        mn = jnp.min(jnp.where(valid, counts, BIGI32))
        min_count = jnp.maximum(jnp.int32(1), mn)
        kv_ref[...] = jnp.minimum(min_count, counts)


_SC_MESH = plsc.VectorSubcoreMesh(core_axis_name="c", subcore_axis_name="s")
CHUNK = B // 16
NV = CHUNK // 16


@functools.partial(
    pl.kernel,
    mesh=_SC_MESH,
    compiler_params=pltpu.CompilerParams(needs_layout_passes=False),
    out_type=jax.ShapeDtypeStruct((B,), jnp.int32),
    scratch_types=[
        pltpu.VMEM((CHUNK,), jnp.int32),
        pltpu.VMEM((CHUNK,), jnp.int32),
        pltpu.VMEM((CHUNK,), jnp.int32),
        pltpu.VMEM((4096,), jnp.int32),
        pltpu.VMEM((128,), jnp.int32),
        pltpu.VMEM((128,), jnp.int32),
        pltpu.VMEM((16,), jnp.int32),
        pltpu.VMEM((16,), jnp.int32),
        pltpu.VMEM((16, 512), jnp.int32),
        pltpu.VMEM((512,), jnp.int32),
        pltpu.VMEM((16,), jnp.int32),
        pltpu.VMEM_SHARED((128, 512), jnp.int32),
        pltpu.SemaphoreType.DMA,
        pltpu.VMEM_SHARED((128,), jnp.int32),
        pltpu.VMEM_SHARED((128,), jnp.int32),
    ],
)
def _sc_topk(vk_hbm, lab_hbm, kv_hbm, out_hbm,
             vk_v, lab_v, sel_v, hist_v, pref_v, ustar_v, kk_v, prefsl_v,
             acc_v, hsum_v, flagsl_v, shist_sh, dmasem, spref_sh, tflag_sh):
    cid = lax.axis_index("c")
    sid = lax.axis_index("s")

    @pl.when(cid == 0)
    def _():
        base0 = sid * CHUNK
        pltpu.sync_copy(vk_hbm.at[pl.ds(base0, CHUNK)], vk_v)
        pltpu.sync_copy(lab_hbm.at[pl.ds(base0, CHUNK)], lab_v)
        zeros16 = jnp.zeros((16,), jnp.int32)
        ones16 = jnp.ones((16,), jnp.int32)
        iota16 = lax.iota(jnp.int32, 16)

        @pl.when(sid < 8)
        def _():
            pltpu.sync_copy(kv_hbm.at[pl.ds(sid * 16, 16)], kk_v)

        def clearhist(i, _):
            hist_v[pl.ds(i * 16, 16)] = zeros16
            return 0

        def initg(g, _):
            pref_v[pl.ds(g * 16, 16)] = zeros16
            return 0

        lax.fori_loop(0, 8, initg, 0)
        lax.fori_loop(0, 256, clearhist, 0)

        def aggregate_and_scan(ascending, vround=None):
            copies = [
                pltpu.async_copy(hist_v.at[pl.ds(g * 512, 512)],
                                 shist_sh.at[g * 16 + sid], dmasem)
                for g in range(8)
            ]
            for cp in copies:
                cp.wait()
            lax.fori_loop(0, 256, clearhist, 0)
            plsc.subcore_barrier()

            @pl.when(sid < 8)
            def _():
                g = sid
                pltpu.sync_copy(shist_sh.at[pl.ds(g * 16, 16), :], acc_v)

                def sumcol(c, _):
                    def sumt(t, a):
                        return a + acc_v[t, pl.ds(c * 16, 16)]

                    hsum_v[pl.ds(c * 16, 16)] = lax.fori_loop(
                        0, 16, sumt, zeros16)
                    return 0

                lax.fori_loop(0, 32, sumcol, 0)
                kkv = kk_v[...]
                prefg = pref_v[pl.ds(g * 16, 16)]

                def sumb(b, acc):
                    return acc + hsum_v[pl.ds(b * 16, 16)]

                S = lax.fori_loop(0, 32, sumb, zeros16)
                T = kkv if ascending else S - kkv + 1

                def pick(b, carry):
                    C, prevm, bstar, Aat, Cat = carry
                    A = hsum_v[pl.ds(b * 16, 16)]
                    C = C + A
                    m = (C >= T).astype(jnp.int32)
                    d = m - prevm
                    return (C, m, bstar + b * d, Aat + A * d, Cat + C * d)

                init = (zeros16, zeros16, zeros16, zeros16, zeros16)
                _, _, bstar, Aat, Cat = lax.fori_loop(0, 32, pick, init)
                if ascending:
                    kknew = kkv - (Cat - Aat)
                else:
                    kknew = kkv - (S - Cat)
                kk_v[...] = kknew
                prefsl_v[...] = prefg * 32 + bstar
                pltpu.sync_copy(prefsl_v, spref_sh.at[pl.ds(g * 16, 16)])
                if vround is not None:
                    @pl.when(vround == 5)
                    def _():
                        flagsl_v[...] = (kknew < Aat).astype(jnp.int32)
                        pltpu.sync_copy(
                            flagsl_v, tflag_sh.at[pl.ds(g * 16, 16)])

            plsc.subcore_barrier()
            pltpu.sync_copy(spref_sh, pref_v)

        def value_round(r, _):
            s = 25 - 5 * r

            def samp(v, _):
                base = v * 16
                vk16 = vk_v[pl.ds(base, 16)]
                lb16 = lab_v[pl.ds(base, 16)]
                u = vk16 - KEY_BASE
                gate = vk16 >= 0
                pref = plsc.load_gather(pref_v, [lb16])
                active = gate & (lax.shift_right_arithmetic(u, s + 5) == pref)
                binv = lax.shift_right_arithmetic(u, s) & 31
                slot = ((lax.shift_right_logical(lb16, 4) * 512)
                        + binv * 16 + (lb16 & 15))
                plsc.addupdate_scatter(hist_v, [slot], ones16, mask=active)
                return 0

            lax.fori_loop(0, NV, samp, 0)
            aggregate_and_scan(ascending=False, vround=r)
            return 0

        lax.fori_loop(0, 6, value_round, 0)

        def stash(g, _):
            ustar_v[pl.ds(g * 16, 16)] = pref_v[pl.ds(g * 16, 16)]
            pref_v[pl.ds(g * 16, 16)] = zeros16
            return 0

        lax.fori_loop(0, 8, stash, 0)

        def index_round(r, _):
            s = 10 - 5 * r

            def samp(v, _):
                base = v * 16
                vk16 = vk_v[pl.ds(base, 16)]
                lb16 = lab_v[pl.ds(base, 16)]
                u = vk16 - KEY_BASE
                gate = vk16 >= 0
                us = plsc.load_gather(ustar_v, [lb16])
                ip = plsc.load_gather(pref_v, [lb16])
                idxv = base0 + base + iota16
                active = (gate & (u == us)
                          & (lax.shift_right_arithmetic(idxv, s + 5) == ip))
                binv = lax.shift_right_arithmetic(idxv, s) & 31
                slot = ((lax.shift_right_logical(lb16, 4) * 512)
                        + binv * 16 + (lb16 & 15))
                plsc.addupdate_scatter(hist_v, [slot], ones16, mask=active)
                return 0

            lax.fori_loop(0, NV, samp, 0)
            aggregate_and_scan(ascending=True)
            return 0

        pltpu.sync_copy(tflag_sh, hsum_v.at[pl.ds(0, 128)])

        def orrow(i, a):
            return a | hsum_v[pl.ds(i * 16, 16)]

        any_tie = lax.reduce_max(lax.fori_loop(0, 8, orrow, zeros16),
                                 axes=(0,))

        def run_index(_):
            lax.fori_loop(0, 3, index_round, 0)
            return 0

        def skip_index(_):
            def setbig(g2, _):
                pref_v[pl.ds(g2 * 16, 16)] = jnp.broadcast_to(
                    jnp.int32(1 << 20), (16,))
                return 0

            lax.fori_loop(0, 8, setbig, 0)
            return 0

        lax.cond(any_tie > 0, run_index, skip_index, 0)

        def fin(v, _):
            base = v * 16
            vk16 = vk_v[pl.ds(base, 16)]
            lb16 = lab_v[pl.ds(base, 16)]
            u = vk16 - KEY_BASE
            gate = vk16 >= 0
            us = plsc.load_gather(ustar_v, [lb16])
            ms = plsc.load_gather(pref_v, [lb16])
            idxv = base0 + base + iota16
            sel = gate & ((u > us) | ((u == us) & (idxv <= ms)))
            sel_v[pl.ds(base, 16)] = sel.astype(jnp.int32)
            return 0

        lax.fori_loop(0, NV, fin, 0)
        pltpu.sync_copy(sel_v, out_hbm.at[pl.ds(base0, CHUNK)])


def kernel(logits):
    maxp, lab, vk, kvec = pl.pallas_call(
        _stage1_body,
        grid=(B // R1,),
        in_specs=[pl.BlockSpec((R1, NUM_CLASSES), lambda i: (i, 0))],
        out_specs=[
            pl.BlockSpec((R1, 1), lambda i: (i, 0)),
            pl.BlockSpec((R1, 1), lambda i: (i, 0)),
            pl.BlockSpec((R1, 1), lambda i: (i, 0)),
            pl.BlockSpec((1, 128), lambda i: (0, 0)),
        ],
        out_shape=[
            jax.ShapeDtypeStruct((B, 1), jnp.float32),
            jax.ShapeDtypeStruct((B, 1), jnp.int32),
            jax.ShapeDtypeStruct((B, 1), jnp.int32),
            jax.ShapeDtypeStruct((1, 128), jnp.int32),
        ],
    )(logits)

    sel = _sc_topk(jnp.reshape(vk, (B,)), jnp.reshape(lab, (B,)),
                   jnp.reshape(kvec, (128,)))

    pseudo_labels = jnp.reshape(lab, (B,))
    confidence_mask = sel.astype(bool)
    max_probs = jnp.reshape(maxp, (B,))
    return (pseudo_labels, confidence_mask, max_probs)


# speedup vs baseline: 1.1103x; 1.1103x over previous
import functools

import jax
import jax.numpy as jnp
from jax import lax
from jax.experimental import pallas as pl
from jax.experimental.pallas import tpu as pltpu
from jax.experimental.pallas import tpu_sc as plsc

NUM_CLASSES = 100
THRESH = 0.05
B = 16384
R1 = 2048
BIGI32 = 2**30
KEY_BASE = 0x3C000000


def _stage1_body(x_ref, maxp_ref, lab_ref, vk_ref, kv_ref):
    step = pl.program_id(0)
    x = x_ref[...]
    m = jnp.max(x, axis=1, keepdims=True)
    e = jnp.exp(x - m)
    s = jnp.sum(e, axis=1, keepdims=True)
    p = e / s
    maxp = jnp.max(p, axis=1, keepdims=True)
    iotaf = jax.lax.broadcasted_iota(jnp.int32, p.shape, 1).astype(jnp.float32)
    labf = jnp.min(jnp.where(p >= maxp, iotaf, jnp.float32(1e9)),
                   axis=1, keepdims=True)
    lab = labf.astype(jnp.int32)
    conf = maxp >= THRESH
    vk = jnp.where(conf, jax.lax.bitcast_convert_type(maxp, jnp.int32),
                   jnp.int32(-1))
    maxp_ref[...] = maxp
    lab_ref[...] = lab
    vk_ref[...] = vk
    lane = jax.lax.broadcasted_iota(jnp.int32, (R1, 128), 1)
    onehot = (lab == lane) & conf
    cnt = jnp.sum(onehot.astype(jnp.int32), axis=0, keepdims=True)

    @pl.when(step == 0)
    def _():
        kv_ref[...] = cnt

    @pl.when(step > 0)
    def _():
        kv_ref[...] += cnt

    @pl.when(step == pl.num_programs(0) - 1)
    def _():
        counts = kv_ref[...]
        lane1 = jax.lax.broadcasted_iota(jnp.int32, (1, 128), 1)
        valid = lane1 < NUM_CLASSES
        mn = jnp.min(jnp.where(valid, counts, BIGI32))
        min_count = jnp.maximum(jnp.int32(1), mn)
        kv_ref[...] = jnp.minimum(min_count, counts)


_SC_MESH = plsc.VectorSubcoreMesh(core_axis_name="c", subcore_axis_name="s")
CHUNK = B // 16
NV = CHUNK // 16


@functools.partial(
    pl.kernel,
    mesh=_SC_MESH,
    compiler_params=pltpu.CompilerParams(needs_layout_passes=False),
    out_type=jax.ShapeDtypeStruct((B,), jnp.int32),
    scratch_types=[
        pltpu.VMEM((CHUNK,), jnp.int32),
        pltpu.VMEM((CHUNK,), jnp.int32),
        pltpu.VMEM((CHUNK,), jnp.int32),
        pltpu.VMEM((2048,), jnp.int32),
        pltpu.VMEM((128,), jnp.int32),
        pltpu.VMEM((128,), jnp.int32),
        pltpu.VMEM((16,), jnp.int32),
        pltpu.VMEM((16,), jnp.int32),
        pltpu.VMEM((16, 256), jnp.int32),
        pltpu.VMEM((256,), jnp.int32),
        pltpu.VMEM((16,), jnp.int32),
        pltpu.VMEM_SHARED((128, 256), jnp.int32),
        pltpu.SemaphoreType.DMA,
        pltpu.VMEM_SHARED((128,), jnp.int32),
        pltpu.VMEM_SHARED((128,), jnp.int32),
    ],
)
def _sc_topk(vk_hbm, lab_hbm, kv_hbm, out_hbm,
             vk_v, lab_v, sel_v, hist_v, pref_v, ustar_v, kk_v, prefsl_v,
             acc_v, hsum_v, flagsl_v, shist_sh, dmasem, spref_sh, tflag_sh):
    cid = lax.axis_index("c")
    sid = lax.axis_index("s")

    @pl.when(cid == 0)
    def _():
        base0 = sid * CHUNK
        pltpu.sync_copy(vk_hbm.at[pl.ds(base0, CHUNK)], vk_v)
        pltpu.sync_copy(lab_hbm.at[pl.ds(base0, CHUNK)], lab_v)
        zeros16 = jnp.zeros((16,), jnp.int32)
        ones16 = jnp.ones((16,), jnp.int32)
        iota16 = lax.iota(jnp.int32, 16)

        @pl.when(sid < 8)
        def _():
            pltpu.sync_copy(kv_hbm.at[pl.ds(sid * 16, 16)], kk_v)

        def clearhist(i, _):
            hist_v[pl.ds(i * 16, 16)] = zeros16
            return 0

        def initg(g, _):
            pref_v[pl.ds(g * 16, 16)] = zeros16
            return 0

        lax.fori_loop(0, 8, initg, 0)
        lax.fori_loop(0, 128, clearhist, 0)

        def aggregate_and_scan(ascending, vround=None):
            copies = [
                pltpu.async_copy(hist_v.at[pl.ds(g * 256, 256)],
                                 shist_sh.at[g * 16 + sid], dmasem)
                for g in range(8)
            ]
            for cp in copies:
                cp.wait()
            lax.fori_loop(0, 128, clearhist, 0)
            plsc.subcore_barrier()

            @pl.when(sid < 8)
            def _():
                g = sid
                pltpu.sync_copy(shist_sh.at[pl.ds(g * 16, 16), :], acc_v)

                def sumcol(c, _):
                    def sumt(t, a):
                        return a + acc_v[t, pl.ds(c * 16, 16)]

                    hsum_v[pl.ds(c * 16, 16)] = lax.fori_loop(
                        0, 16, sumt, zeros16)
                    return 0

                lax.fori_loop(0, 16, sumcol, 0)
                kkv = kk_v[...]
                prefg = pref_v[pl.ds(g * 16, 16)]

                def sumb(b, acc):
                    return acc + hsum_v[pl.ds(b * 16, 16)]

                S = lax.fori_loop(0, 16, sumb, zeros16)
                T = kkv if ascending else S - kkv + 1

                def pick(b, carry):
                    C, prevm, bstar, Aat, Cat = carry
                    A = hsum_v[pl.ds(b * 16, 16)]
                    C = C + A
                    m = (C >= T).astype(jnp.int32)
                    d = m - prevm
                    return (C, m, bstar + b * d, Aat + A * d, Cat + C * d)

                init = (zeros16, zeros16, zeros16, zeros16, zeros16)
                _, _, bstar, Aat, Cat = lax.fori_loop(0, 16, pick, init)
                if ascending:
                    kknew = kkv - (Cat - Aat)
                else:
                    kknew = kkv - (S - Cat)
                kk_v[...] = kknew
                prefsl_v[...] = prefg * 16 + bstar
                pltpu.sync_copy(prefsl_v, spref_sh.at[pl.ds(g * 16, 16)])
                if vround is not None:
                    @pl.when(vround == 6)
                    def _():
                        flagsl_v[...] = (kknew < Aat).astype(jnp.int32)
                        pltpu.sync_copy(
                            flagsl_v, tflag_sh.at[pl.ds(g * 16, 16)])

            plsc.subcore_barrier()
            pltpu.sync_copy(spref_sh, pref_v)

        def value_round(r, _):
            s = 24 - 4 * r

            def samp(v, _):
                base = v * 16
                vk16 = vk_v[pl.ds(base, 16)]
                lb16 = lab_v[pl.ds(base, 16)]
                u = vk16 - KEY_BASE
                gate = vk16 >= 0
                pref = plsc.load_gather(pref_v, [lb16])
                active = gate & (lax.shift_right_arithmetic(u, s + 4) == pref)
                binv = lax.shift_right_arithmetic(u, s) & 15
                slot = ((lax.shift_right_logical(lb16, 4) * 256)
                        + binv * 16 + (lb16 & 15))
                plsc.addupdate_scatter(hist_v, [slot], ones16, mask=active)
                return 0

            lax.fori_loop(0, NV, samp, 0)
            aggregate_and_scan(ascending=False, vround=r)
            return 0

        lax.fori_loop(0, 7, value_round, 0)

        def stash(g, _):
            ustar_v[pl.ds(g * 16, 16)] = pref_v[pl.ds(g * 16, 16)]
            pref_v[pl.ds(g * 16, 16)] = zeros16
            return 0

        lax.fori_loop(0, 8, stash, 0)

        def index_round(r, _):
            s = 12 - 4 * r

            def samp(v, _):
                base = v * 16
                vk16 = vk_v[pl.ds(base, 16)]
                lb16 = lab_v[pl.ds(base, 16)]
                u = vk16 - KEY_BASE
                gate = vk16 >= 0
                us = plsc.load_gather(ustar_v, [lb16])
                ip = plsc.load_gather(pref_v, [lb16])
                idxv = base0 + base + iota16
                active = (gate & (u == us)
                          & (lax.shift_right_arithmetic(idxv, s + 4) == ip))
                binv = lax.shift_right_arithmetic(idxv, s) & 15
                slot = ((lax.shift_right_logical(lb16, 4) * 256)
                        + binv * 16 + (lb16 & 15))
                plsc.addupdate_scatter(hist_v, [slot], ones16, mask=active)
                return 0

            lax.fori_loop(0, NV, samp, 0)
            aggregate_and_scan(ascending=True)
            return 0

        pltpu.sync_copy(tflag_sh, hsum_v.at[pl.ds(0, 128)])

        def orrow(i, a):
            return a | hsum_v[pl.ds(i * 16, 16)]

        any_tie = lax.reduce_max(lax.fori_loop(0, 8, orrow, zeros16),
                                 axes=(0,))

        def run_index(_):
            lax.fori_loop(0, 4, index_round, 0)
            return 0

        def skip_index(_):
            def setbig(g2, _):
                pref_v[pl.ds(g2 * 16, 16)] = jnp.broadcast_to(
                    jnp.int32(1 << 20), (16,))
                return 0

            lax.fori_loop(0, 8, setbig, 0)
            return 0

        lax.cond(any_tie > 0, run_index, skip_index, 0)

        def fin(v, _):
            base = v * 16
            vk16 = vk_v[pl.ds(base, 16)]
            lb16 = lab_v[pl.ds(base, 16)]
            u = vk16 - KEY_BASE
            gate = vk16 >= 0
            us = plsc.load_gather(ustar_v, [lb16])
            ms = plsc.load_gather(pref_v, [lb16])
            idxv = base0 + base + iota16
            sel = gate & ((u > us) | ((u == us) & (idxv <= ms)))
            sel_v[pl.ds(base, 16)] = sel.astype(jnp.int32)
            return 0

        lax.fori_loop(0, NV, fin, 0)
        pltpu.sync_copy(sel_v, out_hbm.at[pl.ds(base0, CHUNK)])


def kernel(logits):
    maxp, lab, vk, kvec = pl.pallas_call(
        _stage1_body,
        grid=(B // R1,),
        in_specs=[pl.BlockSpec((R1, NUM_CLASSES), lambda i: (i, 0))],
        out_specs=[
            pl.BlockSpec((R1, 1), lambda i: (i, 0)),
            pl.BlockSpec((R1, 1), lambda i: (i, 0)),
            pl.BlockSpec((R1, 1), lambda i: (i, 0)),
            pl.BlockSpec((1, 128), lambda i: (0, 0)),
        ],
        out_shape=[
            jax.ShapeDtypeStruct((B, 1), jnp.float32),
            jax.ShapeDtypeStruct((B, 1), jnp.int32),
            jax.ShapeDtypeStruct((B, 1), jnp.int32),
            jax.ShapeDtypeStruct((1, 128), jnp.int32),
        ],
    )(logits)

    sel = _sc_topk(jnp.reshape(vk, (B,)), jnp.reshape(lab, (B,)),
                   jnp.reshape(kvec, (128,)))

    pseudo_labels = jnp.reshape(lab, (B,))
    confidence_mask = sel.astype(bool)
    max_probs = jnp.reshape(maxp, (B,))
    return (pseudo_labels, confidence_mask, max_probs)
